# baseline (speedup 1.0000x reference)
import functools

import jax
import jax.numpy as jnp
from jax import lax
from jax.experimental import pallas as pl
from jax.experimental.pallas import tpu as pltpu
from jax.experimental.pallas import tpu_sc as plsc

D_MODEL = 64
K = 16
NC = 2
NS = 16
NW = NC * NS
CHUNK = 128


def _make_sc_gather(n_rows):
    assert n_rows % NW == 0
    b_per_w = n_rows // NW
    assert b_per_w % CHUNK == 0
    n_chunks = b_per_w // CHUNK
    mesh = plsc.VectorSubcoreMesh(core_axis_name="c", subcore_axis_name="s")

    @functools.partial(
        pl.kernel,
        out_type=jax.ShapeDtypeStruct((n_rows, 128), jnp.float32),
        mesh=mesh,
        scratch_types=[
            pltpu.VMEM((b_per_w,), jnp.int32),
            pltpu.VMEM((CHUNK, 128), jnp.float32),
            pltpu.SemaphoreType.DMA,
        ],
    )
    def gather(idx_hbm, table_hbm, out_hbm, idx_v, rows_v, sem):
        wid = lax.axis_index("s") * NC + lax.axis_index("c")
        base = wid * b_per_w
        pltpu.sync_copy(idx_hbm.at[pl.ds(base, b_per_w)], idx_v)

        def body(c, carry):
            pltpu.async_copy(
                table_hbm.at[idx_v.at[pl.ds(c * CHUNK, CHUNK)]], rows_v, sem
            ).wait()
            pltpu.sync_copy(rows_v, out_hbm.at[pl.ds(base + c * CHUNK, CHUNK)])
            return carry

        lax.fori_loop(0, n_chunks, body, 0)

    return gather


def _matmul(emb128, parity, Vk):
    m = emb128.shape[0]
    bm = 2048
    assert m % bm == 0

    def mm(x_ref, p_ref, vk_ref, o_ref):
        lo = jnp.dot(
            x_ref[:, :D_MODEL], vk_ref[...], preferred_element_type=jnp.float32
        )
        hi = jnp.dot(
            x_ref[:, D_MODEL:], vk_ref[...], preferred_element_type=jnp.float32
        )
        p = p_ref[...]
        o_ref[...] = lo + p * (hi - lo)

    return pl.pallas_call(
        mm,
        grid=(m // bm,),
        in_specs=[
            pl.BlockSpec((bm, 128), lambda i: (i, 0)),
            pl.BlockSpec((bm, 1), lambda i: (i, 0)),
            pl.BlockSpec((D_MODEL, K), lambda i: (0, 0)),
        ],
        out_specs=pl.BlockSpec((bm, K), lambda i: (i, 0)),
        out_shape=jax.ShapeDtypeStruct((m, K), jnp.float32),
    )(emb128, parity, Vk)


def kernel(input_ids, table, Vk):
    b, l = input_ids.shape
    n = b * l
    idx = input_ids.reshape(n).astype(jnp.int32)
    table_pairs = table.reshape(n_pairs := table.shape[0] // 2, 128)
    del n_pairs
    idx2 = idx >> 1
    parity = (idx & 1).astype(jnp.float32).reshape(n, 1)
    emb128 = _make_sc_gather(n)(idx2, table_pairs)
    out = _matmul(emb128, parity, Vk)
    return out.reshape(b, l, K)


# speedup vs baseline: 1.6771x; 1.6771x over previous
import functools

import jax
import jax.numpy as jnp
from jax import lax
from jax.experimental import pallas as pl
from jax.experimental.pallas import tpu as pltpu
from jax.experimental.pallas import tpu_sc as plsc

D_MODEL = 64
K = 16
NC = 2
NS = 16
NW = NC * NS
BATCH = 128
LINES = BATCH // 2


def _make_sc_gather(n_rows):
    assert n_rows % (NW * BATCH) == 0
    b_per_w = n_rows // NW
    n_batches = b_per_w // BATCH
    l_per_w = b_per_w // 2
    mesh = plsc.VectorSubcoreMesh(core_axis_name="c", subcore_axis_name="s")

    @functools.partial(
        pl.kernel,
        out_type=jax.ShapeDtypeStruct((n_rows // 2, 128), jnp.float32),
        mesh=mesh,
        scratch_types=[
            pltpu.VMEM((b_per_w,), jnp.int32),
            pltpu.VMEM((LINES, 128), jnp.float32),
            pltpu.VMEM((LINES, 128), jnp.float32),
            pltpu.SemaphoreType.DMA,
            pltpu.SemaphoreType.DMA,
        ],
    )
    def gather(idx_hbm, table_hbm, out_hbm, idx_v, rows0, rows1, sem0, sem1):
        wid = lax.axis_index("s") * NC + lax.axis_index("c")
        base = wid * b_per_w
        lbase = wid * l_per_w
        pltpu.sync_copy(idx_hbm.at[pl.ds(base, b_per_w)], idx_v)
        bufs = (rows0, rows1)
        sems = (sem0, sem1)

        def fire(b, parity):
            buf, sem = bufs[parity], sems[parity]
            for g in range(BATCH // 16):
                iv = idx_v[pl.ds(b * BATCH + g * 16, 16)]
                for u in range(16):
                    t = g * 16 + u
                    pltpu.async_copy(
                        table_hbm.at[iv[u]],
                        buf.at[t // 2, pl.ds((t % 2) * D_MODEL, D_MODEL)],
                        sem,
                    )

        def drain_and_flush(b, parity):
            buf, sem = bufs[parity], sems[parity]
            pltpu.make_async_copy(out_hbm.at[pl.ds(0, LINES)], buf, sem).wait()
            pltpu.sync_copy(buf, out_hbm.at[pl.ds(lbase + b * LINES, LINES)])

        fire(0, 0)

        def pair_body(k, carry):
            b0 = k * 2
            fire_dyn(b0 + 1, 1)
            drain_and_flush(b0, 0)
            fire_dyn(b0 + 2, 0)
            drain_and_flush(b0 + 1, 1)
            return carry

        def fire_dyn(b, parity):
            @pl.when(b < n_batches)
            def _():
                fire(b, parity)

        lax.fori_loop(0, n_batches // 2, pair_body, 0)

    return gather


def _matmul(emb128, Vk):
    n_lines = emb128.shape[0]
    bl = 1600
    assert n_lines % bl == 0
    grid = n_lines // bl

    def mm(x_ref, vk_ref, o_ref):
        lo = jnp.dot(
            x_ref[:, :D_MODEL], vk_ref[...], preferred_element_type=jnp.float32
        )
        hi = jnp.dot(
            x_ref[:, D_MODEL:], vk_ref[...], preferred_element_type=jnp.float32
        )
        y = jnp.stack([lo, hi], axis=1).reshape(2 * bl, K)
        o_ref[...] = y.reshape(o_ref.shape)

    return pl.pallas_call(
        mm,
        grid=(grid,),
        in_specs=[
            pl.BlockSpec((bl, 128), lambda i: (i, 0)),
            pl.BlockSpec((D_MODEL, K), lambda i: (0, 0)),
        ],
        out_specs=pl.BlockSpec((64, 50, K), lambda i: (i, 0, 0)),
        out_shape=jax.ShapeDtypeStruct((grid * 64, 50, K), jnp.float32),
    )(emb128, Vk)


def kernel(input_ids, table, Vk):
    b, l = input_ids.shape
    n = b * l
    idx = input_ids.reshape(n).astype(jnp.int32)
    emb128 = _make_sc_gather(n)(idx, table)
    return _matmul(emb128, Vk)
